# Initial kernel scaffold; baseline (speedup 1.0000x reference)
import functools

import jax
import jax.numpy as jnp
from jax import lax
from jax.experimental import pallas as pl
from jax.experimental.pallas import tpu as pltpu
from jax.experimental.pallas import tpu_sc as plsc

_NC = 2
_NS = 16
_CH = 80
_ZR = 125


def _sc_aggregate(h, src, dst, with_cnt):
    n, d = h.shape
    e = src.shape[0]
    nw = _NC * _NS
    epw = e // nw
    nch = epw // _CH
    rpw = n // _NS
    lanes = d // 16

    out_type = [jax.ShapeDtypeStruct((_NC, n, d), jnp.float32)]
    scratch = [
        pltpu.VMEM((_CH,), jnp.int32),
        pltpu.VMEM((_CH,), jnp.int32),
        pltpu.VMEM((_CH, d), jnp.float32),
        pltpu.VMEM((_ZR, d), jnp.float32),
        pltpu.VMEM_SHARED((n, d), jnp.float32),
        pltpu.SemaphoreType.DMA,
    ]
    if with_cnt:
        out_type.append(jax.ShapeDtypeStruct((_NC, n, 16), jnp.float32))
        scratch += [
            pltpu.VMEM((_CH, 16), jnp.float32),
            pltpu.VMEM_SHARED((n, 16), jnp.float32),
        ]

    def body(h_hbm, src_hbm, dst_hbm, agg_out, *rest):
        if with_cnt:
            (cnt_out, src_v, dst_v, rows_v, zbuf, agg_sh, sem,
             ones_v, cnt_sh) = rest
        else:
            (src_v, dst_v, rows_v, zbuf, agg_sh, sem) = rest
        c = lax.axis_index("c")
        s = lax.axis_index("s")
        wid = c * _NS + s

        def fill_z(i, carry):
            zbuf[i // lanes, pl.ds((i % lanes) * 16, 16)] = jnp.zeros(
                (16,), jnp.float32)
            return carry
        lax.fori_loop(0, _ZR * lanes, fill_z, 0)
        if with_cnt:
            def fill_o(i, carry):
                ones_v[i, :] = jnp.ones((16,), jnp.float32)
                return carry
            lax.fori_loop(0, _CH, fill_o, 0)

        r0 = s * rpw
        for j in range(rpw // _ZR):
            pltpu.sync_copy(zbuf, agg_sh.at[pl.ds(r0 + j * _ZR, _ZR), :])
            if with_cnt:
                pltpu.sync_copy(zbuf.at[:, pl.ds(0, 16)],
                                cnt_sh.at[pl.ds(r0 + j * _ZR, _ZR), :])
        plsc.subcore_barrier()

        def step(i, carry):
            base = wid * epw + i * _CH
            pltpu.sync_copy(src_hbm.at[pl.ds(base, _CH)], src_v)
            pltpu.sync_copy(dst_hbm.at[pl.ds(base, _CH)], dst_v)
            pltpu.async_copy(h_hbm.at[src_v], rows_v, sem).wait()
            pltpu.sync_copy(rows_v, agg_sh.at[dst_v], add=True)
            if with_cnt:
                pltpu.sync_copy(ones_v, cnt_sh.at[dst_v], add=True)
            return carry
        lax.fori_loop(0, nch, step, 0)

        plsc.subcore_barrier()
        pltpu.sync_copy(agg_sh.at[pl.ds(r0, rpw), :],
                        agg_out.at[c, pl.ds(r0, rpw), :])
        if with_cnt:
            pltpu.sync_copy(cnt_sh.at[pl.ds(r0, rpw), :],
                            cnt_out.at[c, pl.ds(r0, rpw), :])

    mesh = plsc.VectorSubcoreMesh(core_axis_name="c", subcore_axis_name="s")
    fn = pl.kernel(body, out_type=out_type, mesh=mesh, scratch_types=scratch)
    return fn(h, src, dst)


_BR = 1000


def _embed(x, w, b):
    n, d = x.shape
    h = w.shape[1]

    def body(x_ref, w_ref, b_ref, o_ref):
        o_ref[...] = jnp.dot(x_ref[...], w_ref[...],
                             preferred_element_type=jnp.float32) + b_ref[...]

    return pl.pallas_call(
        body,
        grid=(n // _BR,),
        in_specs=[
            pl.BlockSpec((_BR, d), lambda i: (i, 0)),
            pl.BlockSpec((d, h), lambda i: (0, 0)),
            pl.BlockSpec((1, h), lambda i: (0, 0)),
        ],
        out_specs=pl.BlockSpec((_BR, h), lambda i: (i, 0)),
        out_shape=jax.ShapeDtypeStruct((n, h), jnp.float32),
    )(x, w, b.reshape(1, h))


def _sage_dense(aggp, cntp, h_in, wl, bl, wr):
    n, h = h_in.shape
    nb = n // _BR

    def body(a_ref, c_ref, h_ref, wl_ref, bl_ref, wr_ref, t_ref, m_ref):
        i = pl.program_id(0)
        agg = a_ref[0] + a_ref[1]
        cnt = jnp.sum(c_ref[0] + c_ref[1], axis=1) * (1.0 / 16.0)
        inv = 1.0 / jnp.maximum(cnt, 1.0)
        mean = agg * inv[:, None]
        t = (jnp.dot(mean, wl_ref[...], preferred_element_type=jnp.float32)
             + bl_ref[...]
             + jnp.dot(h_ref[...], wr_ref[...],
                       preferred_element_type=jnp.float32))
        t_ref[...] = t
        mom = jnp.concatenate(
            [jnp.sum(t, axis=0)[None], jnp.sum(t * t, axis=0)[None],
             jnp.zeros((6, h), jnp.float32)], axis=0)

        @pl.when(i == 0)
        def _():
            m_ref[...] = mom

        @pl.when(i > 0)
        def _():
            m_ref[...] += mom

    return pl.pallas_call(
        body,
        grid=(nb,),
        in_specs=[
            pl.BlockSpec((_NC, _BR, h), lambda i: (0, i, 0)),
            pl.BlockSpec((_NC, _BR, 16), lambda i: (0, i, 0)),
            pl.BlockSpec((_BR, h), lambda i: (i, 0)),
            pl.BlockSpec((h, h), lambda i: (0, 0)),
            pl.BlockSpec((1, h), lambda i: (0, 0)),
            pl.BlockSpec((h, h), lambda i: (0, 0)),
        ],
        out_specs=[
            pl.BlockSpec((_BR, h), lambda i: (i, 0)),
            pl.BlockSpec((8, h), lambda i: (0, 0)),
        ],
        out_shape=[
            jax.ShapeDtypeStruct((n, h), jnp.float32),
            jax.ShapeDtypeStruct((8, h), jnp.float32),
        ],
    )(aggp, cntp, h_in, wl, bl.reshape(1, h), wr)


def _bn_relu(t, mom, g, be):
    n, h = t.shape
    inv_n = 1.0 / n

    def body(t_ref, m_ref, g_ref, be_ref, o_ref):
        mu = m_ref[0:1, :] * inv_n
        var = m_ref[1:2, :] * inv_n - mu * mu
        scale = g_ref[...] * lax.rsqrt(var + 1e-5)
        o_ref[...] = jnp.maximum((t_ref[...] - mu) * scale + be_ref[...], 0.0)

    return pl.pallas_call(
        body,
        grid=(n // _BR,),
        in_specs=[
            pl.BlockSpec((_BR, h), lambda i: (i, 0)),
            pl.BlockSpec((8, h), lambda i: (0, 0)),
            pl.BlockSpec((1, h), lambda i: (0, 0)),
            pl.BlockSpec((1, h), lambda i: (0, 0)),
        ],
        out_specs=pl.BlockSpec((_BR, h), lambda i: (i, 0)),
        out_shape=jax.ShapeDtypeStruct((n, h), jnp.float32),
    )(t, mom, g.reshape(1, h), be.reshape(1, h))


def _pool_head(hm, batch3, w1, b1, w2p, b2p, num_groups):
    n, h = hm.shape
    nb = n // _BR
    g = num_groups

    def body(h_ref, b_ref, w1_ref, b1_ref, w2_ref, b2_ref, o_ref,
             acc, cacc):
        i = pl.program_id(0)
        bvec = b_ref[0, 0, :]
        seg = lax.broadcasted_iota(jnp.int32, (g, _BR), 0)
        p = (seg == bvec[None, :]).astype(jnp.float32)
        pooled = jnp.dot(p, h_ref[...], preferred_element_type=jnp.float32)
        csum = jnp.broadcast_to(jnp.sum(p, axis=1)[None, :], (8, g))

        @pl.when(i == 0)
        def _():
            acc[...] = pooled
            cacc[...] = csum

        @pl.when(i > 0)
        def _():
            acc[...] += pooled
            cacc[...] += csum

        @pl.when(i == nb - 1)
        def _():
            inv = 1.0 / jnp.maximum(cacc[0:1, :], 1.0)
            pm = acc[...] * inv.reshape(g, 1)
            z = jnp.maximum(
                jnp.dot(pm, w1_ref[...], preferred_element_type=jnp.float32)
                + b1_ref[...], 0.0)
            o_ref[...] = jnp.dot(
                z, w2_ref[...], preferred_element_type=jnp.float32) + b2_ref[...]

    return pl.pallas_call(
        body,
        grid=(nb,),
        in_specs=[
            pl.BlockSpec((_BR, h), lambda i: (i, 0)),
            pl.BlockSpec((1, 1, _BR), lambda i: (i, 0, 0)),
            pl.BlockSpec((h, h), lambda i: (0, 0)),
            pl.BlockSpec((1, h), lambda i: (0, 0)),
            pl.BlockSpec((h, h), lambda i: (0, 0)),
            pl.BlockSpec((1, h), lambda i: (0, 0)),
        ],
        out_specs=pl.BlockSpec((g, h), lambda i: (0, 0)),
        out_shape=jax.ShapeDtypeStruct((g, h), jnp.float32),
        scratch_shapes=[
            pltpu.VMEM((g, h), jnp.float32),
            pltpu.VMEM((8, g), jnp.float32),
        ],
    )(hm, batch3, w1, b1.reshape(1, h), w2p, b2p)


def kernel(x, W_emb, b_emb, Wl1, bl1, Wr1, g1, be1, Wl2, bl2, Wr2, g2, be2,
           Wl3, bl3, Wr3, g3, be3, W1, b1, W2, b2, edge_index, batch):
    n, d = x.shape
    h = W_emb.shape[1]
    c = W2.shape[1]
    src = edge_index[0]
    dst = edge_index[1]

    hh = _embed(x, W_emb, b_emb)

    aggp, cntp = _sc_aggregate(hh, src, dst, with_cnt=True)
    t, mom = _sage_dense(aggp, cntp, hh, Wl1, bl1, Wr1)
    m1 = _bn_relu(t, mom, g1, be1)

    aggp2 = _sc_aggregate(m1, src, dst, with_cnt=False)
    t, mom = _sage_dense(aggp2, cntp, m1, Wl2, bl2, Wr2)
    m2 = _bn_relu(t, mom, g2, be2)

    aggp3 = _sc_aggregate(m2, src, dst, with_cnt=False)
    t, mom = _sage_dense(aggp3, cntp, m2, Wl3, bl3, Wr3)
    m3 = _bn_relu(t, mom, g3, be3)

    num_groups = 512
    nb = n // _BR
    batch3 = batch.reshape(nb, 1, _BR)
    w2p = jnp.zeros((h, h), jnp.float32).at[:, :c].set(W2)
    b2p = jnp.zeros((1, h), jnp.float32).at[0, :c].set(b2)
    out_pad = _pool_head(m3, batch3, W1, b1, w2p, b2p, num_groups)
    out = out_pad[:, :c]
    return (out, m1, m2, m3)


# speedup vs baseline: 4.7063x; 4.7063x over previous
import functools

import jax
import jax.numpy as jnp
from jax import lax
from jax.experimental import pallas as pl
from jax.experimental.pallas import tpu as pltpu
from jax.experimental.pallas import tpu_sc as plsc

_NC = 2
_NS = 16
_CH = 80


def _pad_rows(n):
    q = _NS * _CH
    return ((n + q - 1) // q) * q


def _sc_aggregate(h, src, dst):
    n, d = h.shape
    e = src.shape[0]
    nw = _NC * _NS
    epw = e // nw
    nch = epw // _CH
    np_ = _pad_rows(n)
    rpw = np_ // _NS
    lanes = d // 16

    scratch = [
        pltpu.VMEM((_CH,), jnp.int32),
        pltpu.VMEM((_CH,), jnp.int32),
        pltpu.VMEM((_CH, d), jnp.float32),
        pltpu.VMEM_SHARED((np_, d), jnp.float32),
        pltpu.SemaphoreType.DMA,
    ]

    def body(h_hbm, src_hbm, dst_hbm, agg_out, src_v, dst_v, rows_v,
             agg_sh, sem):
        c = lax.axis_index("c")
        s = lax.axis_index("s")
        wid = c * _NS + s

        def fill_z(i, carry):
            rows_v[i // lanes, pl.ds((i % lanes) * 16, 16)] = jnp.zeros(
                (16,), jnp.float32)
            return carry
        lax.fori_loop(0, _CH * lanes, fill_z, 0)

        r0 = s * rpw
        for j in range(rpw // _CH):
            pltpu.sync_copy(rows_v, agg_sh.at[pl.ds(r0 + j * _CH, _CH), :])
        plsc.subcore_barrier()

        def step(i, carry):
            base = wid * epw + i * _CH
            pltpu.sync_copy(src_hbm.at[pl.ds(base, _CH)], src_v)
            pltpu.sync_copy(dst_hbm.at[pl.ds(base, _CH)], dst_v)
            pltpu.async_copy(h_hbm.at[src_v], rows_v, sem).wait()
            pltpu.sync_copy(rows_v, agg_sh.at[dst_v], add=True)
            return carry
        lax.fori_loop(0, nch, step, 0)

        plsc.subcore_barrier()
        pltpu.sync_copy(agg_sh.at[pl.ds(r0, rpw), :],
                        agg_out.at[c, pl.ds(r0, rpw), :])

    mesh = plsc.VectorSubcoreMesh(core_axis_name="c", subcore_axis_name="s")
    fn = pl.kernel(body,
                   out_type=jax.ShapeDtypeStruct((_NC, np_, d), jnp.float32),
                   mesh=mesh, scratch_types=scratch)
    return fn(h, src, dst)


def _sc_count(dst, n, d):
    e = dst.shape[0]
    nw = _NC * _NS
    epw = e // nw
    nch = epw // _CH
    np_ = _pad_rows(n)
    rpw = np_ // _NS
    lanes = d // 16

    scratch = [
        pltpu.VMEM((_CH,), jnp.int32),
        pltpu.VMEM((_CH, d), jnp.float32),
        pltpu.VMEM_SHARED((np_, d), jnp.float32),
    ]

    def body(dst_hbm, cnt_out, dst_v, ones_v, cnt_sh):
        c = lax.axis_index("c")
        s = lax.axis_index("s")
        wid = c * _NS + s

        def fill(val):
            def f(i, carry):
                ones_v[i // lanes, pl.ds((i % lanes) * 16, 16)] = jnp.full(
                    (16,), val, jnp.float32)
                return carry
            lax.fori_loop(0, _CH * lanes, f, 0)

        fill(0.0)
        r0 = s * rpw
        for j in range(rpw // _CH):
            pltpu.sync_copy(ones_v, cnt_sh.at[pl.ds(r0 + j * _CH, _CH), :])
        fill(1.0)
        plsc.subcore_barrier()

        def step(i, carry):
            base = wid * epw + i * _CH
            pltpu.sync_copy(dst_hbm.at[pl.ds(base, _CH)], dst_v)
            pltpu.sync_copy(ones_v, cnt_sh.at[dst_v], add=True)
            return carry
        lax.fori_loop(0, nch, step, 0)

        plsc.subcore_barrier()
        pltpu.sync_copy(cnt_sh.at[pl.ds(r0, rpw), :],
                        cnt_out.at[c, pl.ds(r0, rpw), :])

    mesh = plsc.VectorSubcoreMesh(core_axis_name="c", subcore_axis_name="s")
    fn = pl.kernel(body,
                   out_type=jax.ShapeDtypeStruct((_NC, np_, d), jnp.float32),
                   mesh=mesh, scratch_types=scratch)
    return fn(dst)


_BR = 1000


def _embed(x, w, b):
    n, d = x.shape
    h = w.shape[1]

    def body(x_ref, w_ref, b_ref, o_ref):
        o_ref[...] = jnp.dot(x_ref[...], w_ref[...],
                             preferred_element_type=jnp.float32) + b_ref[...]

    return pl.pallas_call(
        body,
        grid=(n // _BR,),
        in_specs=[
            pl.BlockSpec((_BR, d), lambda i: (i, 0)),
            pl.BlockSpec((d, h), lambda i: (0, 0)),
            pl.BlockSpec((1, h), lambda i: (0, 0)),
        ],
        out_specs=pl.BlockSpec((_BR, h), lambda i: (i, 0)),
        out_shape=jax.ShapeDtypeStruct((n, h), jnp.float32),
    )(x, w, b.reshape(1, h))


def _sage_dense(aggp, cntp, h_in, wl, bl, wr):
    n, h = h_in.shape
    nb = n // _BR

    def body(a_ref, c_ref, h_ref, wl_ref, bl_ref, wr_ref, t_ref, m_ref):
        i = pl.program_id(0)
        agg = a_ref[0] + a_ref[1]
        cnt = c_ref[0] + c_ref[1]
        mean = agg / jnp.maximum(cnt, 1.0)
        t = (jnp.dot(mean, wl_ref[...], preferred_element_type=jnp.float32)
             + bl_ref[...]
             + jnp.dot(h_ref[...], wr_ref[...],
                       preferred_element_type=jnp.float32))
        t_ref[...] = t
        mom = jnp.concatenate(
            [jnp.sum(t, axis=0)[None], jnp.sum(t * t, axis=0)[None],
             jnp.zeros((6, h), jnp.float32)], axis=0)

        @pl.when(i == 0)
        def _():
            m_ref[...] = mom

        @pl.when(i > 0)
        def _():
            m_ref[...] += mom

    return pl.pallas_call(
        body,
        grid=(nb,),
        in_specs=[
            pl.BlockSpec((_NC, _BR, h), lambda i: (0, i, 0)),
            pl.BlockSpec((_NC, _BR, h), lambda i: (0, i, 0)),
            pl.BlockSpec((_BR, h), lambda i: (i, 0)),
            pl.BlockSpec((h, h), lambda i: (0, 0)),
            pl.BlockSpec((1, h), lambda i: (0, 0)),
            pl.BlockSpec((h, h), lambda i: (0, 0)),
        ],
        out_specs=[
            pl.BlockSpec((_BR, h), lambda i: (i, 0)),
            pl.BlockSpec((8, h), lambda i: (0, 0)),
        ],
        out_shape=[
            jax.ShapeDtypeStruct((n, h), jnp.float32),
            jax.ShapeDtypeStruct((8, h), jnp.float32),
        ],
    )(aggp, cntp, h_in, wl, bl.reshape(1, h), wr)


def _bn_relu(t, mom, g, be):
    n, h = t.shape
    inv_n = 1.0 / n

    def body(t_ref, m_ref, g_ref, be_ref, o_ref):
        mu = m_ref[0:1, :] * inv_n
        var = m_ref[1:2, :] * inv_n - mu * mu
        scale = g_ref[...] * lax.rsqrt(var + 1e-5)
        o_ref[...] = jnp.maximum((t_ref[...] - mu) * scale + be_ref[...], 0.0)

    return pl.pallas_call(
        body,
        grid=(n // _BR,),
        in_specs=[
            pl.BlockSpec((_BR, h), lambda i: (i, 0)),
            pl.BlockSpec((8, h), lambda i: (0, 0)),
            pl.BlockSpec((1, h), lambda i: (0, 0)),
            pl.BlockSpec((1, h), lambda i: (0, 0)),
        ],
        out_specs=pl.BlockSpec((_BR, h), lambda i: (i, 0)),
        out_shape=jax.ShapeDtypeStruct((n, h), jnp.float32),
    )(t, mom, g.reshape(1, h), be.reshape(1, h))


def _pool_head(hm, batch3, w1, b1, w2p, b2p, num_groups):
    n, h = hm.shape
    nb = n // _BR
    g = num_groups

    def body(h_ref, b_ref, w1_ref, b1_ref, w2_ref, b2_ref, o_ref,
             acc, cacc):
        i = pl.program_id(0)
        bvec = b_ref[0, 0, :]
        seg = lax.broadcasted_iota(jnp.int32, (g, _BR), 0)
        p = (seg == bvec[None, :]).astype(jnp.float32)
        pooled = jnp.dot(p, h_ref[...], preferred_element_type=jnp.float32)
        csum = jnp.broadcast_to(jnp.sum(p, axis=1)[None, :], (8, g))

        @pl.when(i == 0)
        def _():
            acc[...] = pooled
            cacc[...] = csum

        @pl.when(i > 0)
        def _():
            acc[...] += pooled
            cacc[...] += csum

        @pl.when(i == nb - 1)
        def _():
            inv = 1.0 / jnp.maximum(cacc[0:1, :], 1.0)
            pm = acc[...] * inv.reshape(g, 1)
            z = jnp.maximum(
                jnp.dot(pm, w1_ref[...], preferred_element_type=jnp.float32)
                + b1_ref[...], 0.0)
            o_ref[...] = jnp.dot(
                z, w2_ref[...], preferred_element_type=jnp.float32) + b2_ref[...]

    return pl.pallas_call(
        body,
        grid=(nb,),
        in_specs=[
            pl.BlockSpec((_BR, h), lambda i: (i, 0)),
            pl.BlockSpec((1, 1, _BR), lambda i: (i, 0, 0)),
            pl.BlockSpec((h, h), lambda i: (0, 0)),
            pl.BlockSpec((1, h), lambda i: (0, 0)),
            pl.BlockSpec((h, h), lambda i: (0, 0)),
            pl.BlockSpec((1, h), lambda i: (0, 0)),
        ],
        out_specs=pl.BlockSpec((g, h), lambda i: (0, 0)),
        out_shape=jax.ShapeDtypeStruct((g, h), jnp.float32),
        scratch_shapes=[
            pltpu.VMEM((g, h), jnp.float32),
            pltpu.VMEM((8, g), jnp.float32),
        ],
    )(hm, batch3, w1, b1.reshape(1, h), w2p, b2p)


def kernel(x, W_emb, b_emb, Wl1, bl1, Wr1, g1, be1, Wl2, bl2, Wr2, g2, be2,
           Wl3, bl3, Wr3, g3, be3, W1, b1, W2, b2, edge_index, batch):
    n, d = x.shape
    h = W_emb.shape[1]
    c = W2.shape[1]
    src = edge_index[0]
    dst = edge_index[1]

    hh = _embed(x, W_emb, b_emb)

    cntp = _sc_count(dst, n, h)
    aggp = _sc_aggregate(hh, src, dst)
    t, mom = _sage_dense(aggp, cntp, hh, Wl1, bl1, Wr1)
    m1 = _bn_relu(t, mom, g1, be1)

    aggp2 = _sc_aggregate(m1, src, dst)
    t, mom = _sage_dense(aggp2, cntp, m1, Wl2, bl2, Wr2)
    m2 = _bn_relu(t, mom, g2, be2)

    aggp3 = _sc_aggregate(m2, src, dst)
    t, mom = _sage_dense(aggp3, cntp, m2, Wl3, bl3, Wr3)
    m3 = _bn_relu(t, mom, g3, be3)

    num_groups = 512
    nb = n // _BR
    batch3 = batch.reshape(nb, 1, _BR)
    w2p = jnp.zeros((h, h), jnp.float32).at[:, :c].set(W2)
    b2p = jnp.zeros((1, h), jnp.float32).at[0, :c].set(b2)
    out_pad = _pool_head(m3, batch3, W1, b1, w2p, b2p, num_groups)
    out = out_pad[:, :c]
    return (out, m1, m2, m3)


# speedup vs baseline: 8.7104x; 1.8508x over previous
import functools

import jax
import jax.numpy as jnp
from jax import lax
from jax.experimental import pallas as pl
from jax.experimental.pallas import tpu as pltpu
from jax.experimental.pallas import tpu_sc as plsc

_NC = 2
_NS = 16
_CH = 80
_SB = 25


def _pad_rows(n):
    q = _NS * _CH
    return ((n + q - 1) // q) * q


def _sc_aggregate(h, src, dst):
    n, d = h.shape
    e = src.shape[0]
    nw = _NC * _NS
    epw = e // nw
    nch = epw // _CH
    np_ = _pad_rows(n)
    rpw = np_ // _NS
    lanes = d // 16

    nsb = nch // _SB

    scratch = [
        pltpu.VMEM((_SB * _CH,), jnp.int32),
        pltpu.VMEM((_SB, _CH), jnp.int32),
        pltpu.VMEM((_CH, d), jnp.float32),
        pltpu.VMEM((_CH, d), jnp.float32),
        pltpu.VMEM_SHARED((np_, d), jnp.float32),
        pltpu.SemaphoreType.DMA,
        pltpu.SemaphoreType.DMA,
    ]

    def body(h_hbm, src_hbm, dst_hbm, agg_out, src_v, dst_v, buf0, buf1,
             agg_sh, sg0, sg1):
        c = lax.axis_index("c")
        s = lax.axis_index("s")
        wid = c * _NS + s
        bufs = (buf0, buf1)
        sgs = (sg0, sg1)

        def fill_z(i, carry):
            buf0[i // lanes, pl.ds((i % lanes) * 16, 16)] = jnp.zeros(
                (16,), jnp.float32)
            return carry
        lax.fori_loop(0, _CH * lanes, fill_z, 0)

        r0 = s * rpw
        for j in range(rpw // _CH):
            pltpu.sync_copy(buf0, agg_sh.at[pl.ds(r0 + j * _CH, _CH), :])
        plsc.subcore_barrier()

        def super_step(k, carry):
            pltpu.sync_copy(src_hbm.at[wid, k], src_v)
            pltpu.sync_copy(dst_hbm.at[wid, k], dst_v)
            g0 = pltpu.make_async_copy(
                h_hbm.at[src_v.at[pl.ds(0, _CH)]], buf0, sg0)
            g0.start()
            g1 = pltpu.make_async_copy(
                h_hbm.at[src_v.at[pl.ds(_CH, _CH)]], buf1, sg1)
            g1.start()
            for j in range(_SB):
                p = j % 2
                pltpu.make_async_copy(
                    h_hbm.at[src_v.at[pl.ds(j * _CH, _CH)]], bufs[p],
                    sgs[p]).wait()
                pltpu.sync_copy(bufs[p], agg_sh.at[dst_v.at[j]], add=True)
                if j + 2 < _SB:
                    nxt = pltpu.make_async_copy(
                        h_hbm.at[src_v.at[pl.ds((j + 2) * _CH, _CH)]],
                        bufs[p], sgs[p])
                    nxt.start()
            return carry
        lax.fori_loop(0, nsb, super_step, 0)

        plsc.subcore_barrier()
        pltpu.sync_copy(agg_sh.at[pl.ds(r0, rpw), :],
                        agg_out.at[c, pl.ds(r0, rpw), :])

    mesh = plsc.VectorSubcoreMesh(core_axis_name="c", subcore_axis_name="s")
    fn = pl.kernel(body,
                   out_type=jax.ShapeDtypeStruct((_NC, np_, d), jnp.float32),
                   mesh=mesh, scratch_types=scratch)
    nw_ = _NC * _NS
    srcr = src.reshape(nw_, nsb, _SB * _CH)
    dstr = dst.reshape(nw_, nsb, _SB, _CH)
    return fn(h, srcr, dstr)


def _sc_count(dst, n, d):
    e = dst.shape[0]
    nw = _NC * _NS
    epw = e // nw
    nch = epw // _CH
    np_ = _pad_rows(n)
    rpw = np_ // _NS
    lanes = d // 16

    scratch = [
        pltpu.VMEM((_CH,), jnp.int32),
        pltpu.VMEM((_CH, d), jnp.float32),
        pltpu.VMEM_SHARED((np_, d), jnp.float32),
    ]

    def body(dst_hbm, cnt_out, dst_v, ones_v, cnt_sh):
        c = lax.axis_index("c")
        s = lax.axis_index("s")
        wid = c * _NS + s

        def fill(val):
            def f(i, carry):
                ones_v[i // lanes, pl.ds((i % lanes) * 16, 16)] = jnp.full(
                    (16,), val, jnp.float32)
                return carry
            lax.fori_loop(0, _CH * lanes, f, 0)

        fill(0.0)
        r0 = s * rpw
        for j in range(rpw // _CH):
            pltpu.sync_copy(ones_v, cnt_sh.at[pl.ds(r0 + j * _CH, _CH), :])
        fill(1.0)
        plsc.subcore_barrier()

        def step(i, carry):
            base = wid * epw + i * _CH
            pltpu.sync_copy(dst_hbm.at[pl.ds(base, _CH)], dst_v)
            pltpu.sync_copy(ones_v, cnt_sh.at[dst_v], add=True)
            return carry
        lax.fori_loop(0, nch, step, 0)

        plsc.subcore_barrier()
        pltpu.sync_copy(cnt_sh.at[pl.ds(r0, rpw), :],
                        cnt_out.at[c, pl.ds(r0, rpw), :])

    mesh = plsc.VectorSubcoreMesh(core_axis_name="c", subcore_axis_name="s")
    fn = pl.kernel(body,
                   out_type=jax.ShapeDtypeStruct((_NC, np_, d), jnp.float32),
                   mesh=mesh, scratch_types=scratch)
    return fn(dst)


_BR = 1000


def _embed(x, w, b):
    n, d = x.shape
    h = w.shape[1]

    def body(x_ref, w_ref, b_ref, o_ref):
        o_ref[...] = jnp.dot(x_ref[...], w_ref[...],
                             preferred_element_type=jnp.float32) + b_ref[...]

    return pl.pallas_call(
        body,
        grid=(n // _BR,),
        in_specs=[
            pl.BlockSpec((_BR, d), lambda i: (i, 0)),
            pl.BlockSpec((d, h), lambda i: (0, 0)),
            pl.BlockSpec((1, h), lambda i: (0, 0)),
        ],
        out_specs=pl.BlockSpec((_BR, h), lambda i: (i, 0)),
        out_shape=jax.ShapeDtypeStruct((n, h), jnp.float32),
    )(x, w, b.reshape(1, h))


def _sage_dense(aggp, cntp, h_in, wl, bl, wr):
    n, h = h_in.shape
    nb = n // _BR

    def body(a_ref, c_ref, h_ref, wl_ref, bl_ref, wr_ref, t_ref, m_ref):
        i = pl.program_id(0)
        agg = a_ref[0] + a_ref[1]
        cnt = c_ref[0] + c_ref[1]
        mean = agg / jnp.maximum(cnt, 1.0)
        t = (jnp.dot(mean, wl_ref[...], preferred_element_type=jnp.float32)
             + bl_ref[...]
             + jnp.dot(h_ref[...], wr_ref[...],
                       preferred_element_type=jnp.float32))
        t_ref[...] = t
        mom = jnp.concatenate(
            [jnp.sum(t, axis=0)[None], jnp.sum(t * t, axis=0)[None],
             jnp.zeros((6, h), jnp.float32)], axis=0)

        @pl.when(i == 0)
        def _():
            m_ref[...] = mom

        @pl.when(i > 0)
        def _():
            m_ref[...] += mom

    return pl.pallas_call(
        body,
        grid=(nb,),
        in_specs=[
            pl.BlockSpec((_NC, _BR, h), lambda i: (0, i, 0)),
            pl.BlockSpec((_NC, _BR, h), lambda i: (0, i, 0)),
            pl.BlockSpec((_BR, h), lambda i: (i, 0)),
            pl.BlockSpec((h, h), lambda i: (0, 0)),
            pl.BlockSpec((1, h), lambda i: (0, 0)),
            pl.BlockSpec((h, h), lambda i: (0, 0)),
        ],
        out_specs=[
            pl.BlockSpec((_BR, h), lambda i: (i, 0)),
            pl.BlockSpec((8, h), lambda i: (0, 0)),
        ],
        out_shape=[
            jax.ShapeDtypeStruct((n, h), jnp.float32),
            jax.ShapeDtypeStruct((8, h), jnp.float32),
        ],
    )(aggp, cntp, h_in, wl, bl.reshape(1, h), wr)


def _bn_relu(t, mom, g, be):
    n, h = t.shape
    inv_n = 1.0 / n

    def body(t_ref, m_ref, g_ref, be_ref, o_ref):
        mu = m_ref[0:1, :] * inv_n
        var = m_ref[1:2, :] * inv_n - mu * mu
        scale = g_ref[...] * lax.rsqrt(var + 1e-5)
        o_ref[...] = jnp.maximum((t_ref[...] - mu) * scale + be_ref[...], 0.0)

    return pl.pallas_call(
        body,
        grid=(n // _BR,),
        in_specs=[
            pl.BlockSpec((_BR, h), lambda i: (i, 0)),
            pl.BlockSpec((8, h), lambda i: (0, 0)),
            pl.BlockSpec((1, h), lambda i: (0, 0)),
            pl.BlockSpec((1, h), lambda i: (0, 0)),
        ],
        out_specs=pl.BlockSpec((_BR, h), lambda i: (i, 0)),
        out_shape=jax.ShapeDtypeStruct((n, h), jnp.float32),
    )(t, mom, g.reshape(1, h), be.reshape(1, h))


def _pool_head(hm, batch3, w1, b1, w2p, b2p, num_groups):
    n, h = hm.shape
    nb = n // _BR
    g = num_groups

    def body(h_ref, b_ref, w1_ref, b1_ref, w2_ref, b2_ref, o_ref,
             acc, cacc):
        i = pl.program_id(0)
        bvec = b_ref[0, 0, :]
        seg = lax.broadcasted_iota(jnp.int32, (g, _BR), 0)
        p = (seg == bvec[None, :]).astype(jnp.float32)
        pooled = jnp.dot(p, h_ref[...], preferred_element_type=jnp.float32)
        csum = jnp.broadcast_to(jnp.sum(p, axis=1)[None, :], (8, g))

        @pl.when(i == 0)
        def _():
            acc[...] = pooled
            cacc[...] = csum

        @pl.when(i > 0)
        def _():
            acc[...] += pooled
            cacc[...] += csum

        @pl.when(i == nb - 1)
        def _():
            inv = 1.0 / jnp.maximum(cacc[0:1, :], 1.0)
            pm = acc[...] * inv.reshape(g, 1)
            z = jnp.maximum(
                jnp.dot(pm, w1_ref[...], preferred_element_type=jnp.float32)
                + b1_ref[...], 0.0)
            o_ref[...] = jnp.dot(
                z, w2_ref[...], preferred_element_type=jnp.float32) + b2_ref[...]

    return pl.pallas_call(
        body,
        grid=(nb,),
        in_specs=[
            pl.BlockSpec((_BR, h), lambda i: (i, 0)),
            pl.BlockSpec((1, 1, _BR), lambda i: (i, 0, 0)),
            pl.BlockSpec((h, h), lambda i: (0, 0)),
            pl.BlockSpec((1, h), lambda i: (0, 0)),
            pl.BlockSpec((h, h), lambda i: (0, 0)),
            pl.BlockSpec((1, h), lambda i: (0, 0)),
        ],
        out_specs=pl.BlockSpec((g, h), lambda i: (0, 0)),
        out_shape=jax.ShapeDtypeStruct((g, h), jnp.float32),
        scratch_shapes=[
            pltpu.VMEM((g, h), jnp.float32),
            pltpu.VMEM((8, g), jnp.float32),
        ],
    )(hm, batch3, w1, b1.reshape(1, h), w2p, b2p)


def kernel(x, W_emb, b_emb, Wl1, bl1, Wr1, g1, be1, Wl2, bl2, Wr2, g2, be2,
           Wl3, bl3, Wr3, g3, be3, W1, b1, W2, b2, edge_index, batch):
    n, d = x.shape
    h = W_emb.shape[1]
    c = W2.shape[1]
    src = edge_index[0]
    dst = edge_index[1]

    hh = _embed(x, W_emb, b_emb)

    cntp = _sc_count(dst, n, h)
    aggp = _sc_aggregate(hh, src, dst)
    t, mom = _sage_dense(aggp, cntp, hh, Wl1, bl1, Wr1)
    m1 = _bn_relu(t, mom, g1, be1)

    aggp2 = _sc_aggregate(m1, src, dst)
    t, mom = _sage_dense(aggp2, cntp, m1, Wl2, bl2, Wr2)
    m2 = _bn_relu(t, mom, g2, be2)

    aggp3 = _sc_aggregate(m2, src, dst)
    t, mom = _sage_dense(aggp3, cntp, m2, Wl3, bl3, Wr3)
    m3 = _bn_relu(t, mom, g3, be3)

    num_groups = 512
    nb = n // _BR
    batch3 = batch.reshape(nb, 1, _BR)
    w2p = jnp.zeros((h, h), jnp.float32).at[:, :c].set(W2)
    b2p = jnp.zeros((1, h), jnp.float32).at[0, :c].set(b2)
    out_pad = _pool_head(m3, batch3, W1, b1, w2p, b2p, num_groups)
    out = out_pad[:, :c]
    return (out, m1, m2, m3)


# speedup vs baseline: 9.4663x; 1.0868x over previous
import functools

import jax
import jax.numpy as jnp
from jax import lax
from jax.experimental import pallas as pl
from jax.experimental.pallas import tpu as pltpu
from jax.experimental.pallas import tpu_sc as plsc

_NC = 2
_NS = 16
_CH = 80
_SB = 25


def _pad_rows(n):
    q = _NS * _CH
    return ((n + q - 1) // q) * q


def _sc_aggregate(h, src, dst):
    n, d = h.shape
    e = src.shape[0]
    nw = _NC * _NS
    epw = e // nw
    nch = epw // _CH
    np_ = _pad_rows(n)
    rpw = np_ // _NS
    lanes = d // 16

    nsb = nch // _SB

    scratch = [
        pltpu.VMEM((_SB * _CH,), jnp.int32),
        pltpu.VMEM((_SB, _CH), jnp.int32),
        pltpu.VMEM((_CH, d), jnp.float32),
        pltpu.VMEM((_CH, d), jnp.float32),
        pltpu.VMEM_SHARED((np_, d), jnp.float32),
        pltpu.SemaphoreType.DMA,
        pltpu.SemaphoreType.DMA,
    ]

    def body(h_hbm, src_hbm, dst_hbm, agg_out, src_v, dst_v, buf0, buf1,
             agg_sh, sg0, sg1):
        c = lax.axis_index("c")
        s = lax.axis_index("s")
        wid = c * _NS + s
        bufs = (buf0, buf1)
        sgs = (sg0, sg1)

        def fill_z(i, carry):
            buf0[i // lanes, pl.ds((i % lanes) * 16, 16)] = jnp.zeros(
                (16,), jnp.float32)
            return carry
        lax.fori_loop(0, _CH * lanes, fill_z, 0)

        r0 = s * rpw
        for j in range(rpw // _CH):
            pltpu.sync_copy(buf0, agg_sh.at[pl.ds(r0 + j * _CH, _CH), :])
        plsc.subcore_barrier()

        def super_step(k, carry):
            pltpu.sync_copy(src_hbm.at[wid, k], src_v)
            pltpu.sync_copy(dst_hbm.at[wid, k], dst_v)
            g0 = pltpu.make_async_copy(
                h_hbm.at[src_v.at[pl.ds(0, _CH)]], buf0, sg0)
            g0.start()
            g1 = pltpu.make_async_copy(
                h_hbm.at[src_v.at[pl.ds(_CH, _CH)]], buf1, sg1)
            g1.start()
            for j in range(_SB):
                p = j % 2
                pltpu.make_async_copy(
                    h_hbm.at[src_v.at[pl.ds(j * _CH, _CH)]], bufs[p],
                    sgs[p]).wait()
                pltpu.sync_copy(bufs[p], agg_sh.at[dst_v.at[j]], add=True)
                if j + 2 < _SB:
                    nxt = pltpu.make_async_copy(
                        h_hbm.at[src_v.at[pl.ds((j + 2) * _CH, _CH)]],
                        bufs[p], sgs[p])
                    nxt.start()
            return carry
        lax.fori_loop(0, nsb, super_step, 0)

        plsc.subcore_barrier()
        pltpu.sync_copy(agg_sh.at[pl.ds(r0, rpw), :],
                        agg_out.at[c, pl.ds(r0, rpw), :])

    mesh = plsc.VectorSubcoreMesh(core_axis_name="c", subcore_axis_name="s")
    fn = pl.kernel(body,
                   out_type=jax.ShapeDtypeStruct((_NC, np_, d), jnp.float32),
                   mesh=mesh, scratch_types=scratch)
    nw_ = _NC * _NS
    srcr = src.reshape(nw_, nsb, _SB * _CH)
    dstr = dst.reshape(nw_, nsb, _SB, _CH)
    return fn(h, srcr, dstr)


def _sc_count(dst, n, d):
    e = dst.shape[0]
    nw = _NC * _NS
    epw = e // nw
    nch = epw // _CH
    np_ = _pad_rows(n)
    rpw = np_ // _NS
    lanes = d // 16

    nsb = nch // _SB

    scratch = [
        pltpu.VMEM((_SB, _CH), jnp.int32),
        pltpu.VMEM((_CH, d), jnp.float32),
        pltpu.VMEM_SHARED((np_, d), jnp.float32),
        pltpu.SemaphoreType.DMA,
    ]

    def body(dst_hbm, cnt_out, dst_v, ones_v, cnt_sh, sem):
        c = lax.axis_index("c")
        s = lax.axis_index("s")
        wid = c * _NS + s

        def fill(val):
            def f(i, carry):
                ones_v[i // lanes, pl.ds((i % lanes) * 16, 16)] = jnp.full(
                    (16,), val, jnp.float32)
                return carry
            lax.fori_loop(0, _CH * lanes, f, 0)

        fill(0.0)
        r0 = s * rpw
        for j in range(rpw // _CH):
            pltpu.sync_copy(ones_v, cnt_sh.at[pl.ds(r0 + j * _CH, _CH), :])
        fill(1.0)
        plsc.subcore_barrier()

        def super_step(k, carry):
            pltpu.sync_copy(dst_hbm.at[wid, k], dst_v)
            descs = [pltpu.async_copy(ones_v, cnt_sh.at[dst_v.at[j]], sem,
                                      add=True)
                     for j in range(_SB)]
            for dsc in descs:
                dsc.wait()
            return carry
        lax.fori_loop(0, nsb, super_step, 0)

        plsc.subcore_barrier()
        pltpu.sync_copy(cnt_sh.at[pl.ds(r0, rpw), :],
                        cnt_out.at[c, pl.ds(r0, rpw), :])

    mesh = plsc.VectorSubcoreMesh(core_axis_name="c", subcore_axis_name="s")
    fn = pl.kernel(body,
                   out_type=jax.ShapeDtypeStruct((_NC, np_, d), jnp.float32),
                   mesh=mesh, scratch_types=scratch)
    return fn(dst.reshape(_NC * _NS, nsb, _SB, _CH))


_BR = 1000


def _embed(x, w, b):
    n, d = x.shape
    h = w.shape[1]

    def body(x_ref, w_ref, b_ref, o_ref):
        o_ref[...] = jnp.dot(x_ref[...], w_ref[...],
                             preferred_element_type=jnp.float32) + b_ref[...]

    return pl.pallas_call(
        body,
        grid=(n // _BR,),
        in_specs=[
            pl.BlockSpec((_BR, d), lambda i: (i, 0)),
            pl.BlockSpec((d, h), lambda i: (0, 0)),
            pl.BlockSpec((1, h), lambda i: (0, 0)),
        ],
        out_specs=pl.BlockSpec((_BR, h), lambda i: (i, 0)),
        out_shape=jax.ShapeDtypeStruct((n, h), jnp.float32),
    )(x, w, b.reshape(1, h))


def _sage_dense(aggp, cntp, h_in, wl, bl, wr):
    n, h = h_in.shape
    nb = n // _BR

    def body(a_ref, c_ref, h_ref, wl_ref, bl_ref, wr_ref, t_ref, m_ref):
        i = pl.program_id(0)
        agg = a_ref[0] + a_ref[1]
        cnt = c_ref[0] + c_ref[1]
        mean = agg / jnp.maximum(cnt, 1.0)
        t = (jnp.dot(mean, wl_ref[...], preferred_element_type=jnp.float32)
             + bl_ref[...]
             + jnp.dot(h_ref[...], wr_ref[...],
                       preferred_element_type=jnp.float32))
        t_ref[...] = t
        mom = jnp.concatenate(
            [jnp.sum(t, axis=0)[None], jnp.sum(t * t, axis=0)[None],
             jnp.zeros((6, h), jnp.float32)], axis=0)

        @pl.when(i == 0)
        def _():
            m_ref[...] = mom

        @pl.when(i > 0)
        def _():
            m_ref[...] += mom

    return pl.pallas_call(
        body,
        grid=(nb,),
        in_specs=[
            pl.BlockSpec((_NC, _BR, h), lambda i: (0, i, 0)),
            pl.BlockSpec((_NC, _BR, h), lambda i: (0, i, 0)),
            pl.BlockSpec((_BR, h), lambda i: (i, 0)),
            pl.BlockSpec((h, h), lambda i: (0, 0)),
            pl.BlockSpec((1, h), lambda i: (0, 0)),
            pl.BlockSpec((h, h), lambda i: (0, 0)),
        ],
        out_specs=[
            pl.BlockSpec((_BR, h), lambda i: (i, 0)),
            pl.BlockSpec((8, h), lambda i: (0, 0)),
        ],
        out_shape=[
            jax.ShapeDtypeStruct((n, h), jnp.float32),
            jax.ShapeDtypeStruct((8, h), jnp.float32),
        ],
    )(aggp, cntp, h_in, wl, bl.reshape(1, h), wr)


def _bn_relu(t, mom, g, be):
    n, h = t.shape
    inv_n = 1.0 / n

    def body(t_ref, m_ref, g_ref, be_ref, o_ref):
        mu = m_ref[0:1, :] * inv_n
        var = m_ref[1:2, :] * inv_n - mu * mu
        scale = g_ref[...] * lax.rsqrt(var + 1e-5)
        o_ref[...] = jnp.maximum((t_ref[...] - mu) * scale + be_ref[...], 0.0)

    return pl.pallas_call(
        body,
        grid=(n // _BR,),
        in_specs=[
            pl.BlockSpec((_BR, h), lambda i: (i, 0)),
            pl.BlockSpec((8, h), lambda i: (0, 0)),
            pl.BlockSpec((1, h), lambda i: (0, 0)),
            pl.BlockSpec((1, h), lambda i: (0, 0)),
        ],
        out_specs=pl.BlockSpec((_BR, h), lambda i: (i, 0)),
        out_shape=jax.ShapeDtypeStruct((n, h), jnp.float32),
    )(t, mom, g.reshape(1, h), be.reshape(1, h))


def _pool_head(hm, batch3, w1, b1, w2p, b2p, num_groups):
    n, h = hm.shape
    nb = n // _BR
    g = num_groups

    def body(h_ref, b_ref, w1_ref, b1_ref, w2_ref, b2_ref, o_ref,
             acc, cacc):
        i = pl.program_id(0)
        bvec = b_ref[0, 0, :]
        seg = lax.broadcasted_iota(jnp.int32, (g, _BR), 0)
        p = (seg == bvec[None, :]).astype(jnp.float32)
        pooled = jnp.dot(p, h_ref[...], preferred_element_type=jnp.float32)
        csum = jnp.broadcast_to(jnp.sum(p, axis=1)[None, :], (8, g))

        @pl.when(i == 0)
        def _():
            acc[...] = pooled
            cacc[...] = csum

        @pl.when(i > 0)
        def _():
            acc[...] += pooled
            cacc[...] += csum

        @pl.when(i == nb - 1)
        def _():
            inv = 1.0 / jnp.maximum(cacc[0:1, :], 1.0)
            pm = acc[...] * inv.reshape(g, 1)
            z = jnp.maximum(
                jnp.dot(pm, w1_ref[...], preferred_element_type=jnp.float32)
                + b1_ref[...], 0.0)
            o_ref[...] = jnp.dot(
                z, w2_ref[...], preferred_element_type=jnp.float32) + b2_ref[...]

    return pl.pallas_call(
        body,
        grid=(nb,),
        in_specs=[
            pl.BlockSpec((_BR, h), lambda i: (i, 0)),
            pl.BlockSpec((1, 1, _BR), lambda i: (i, 0, 0)),
            pl.BlockSpec((h, h), lambda i: (0, 0)),
            pl.BlockSpec((1, h), lambda i: (0, 0)),
            pl.BlockSpec((h, h), lambda i: (0, 0)),
            pl.BlockSpec((1, h), lambda i: (0, 0)),
        ],
        out_specs=pl.BlockSpec((g, h), lambda i: (0, 0)),
        out_shape=jax.ShapeDtypeStruct((g, h), jnp.float32),
        scratch_shapes=[
            pltpu.VMEM((g, h), jnp.float32),
            pltpu.VMEM((8, g), jnp.float32),
        ],
    )(hm, batch3, w1, b1.reshape(1, h), w2p, b2p)


def kernel(x, W_emb, b_emb, Wl1, bl1, Wr1, g1, be1, Wl2, bl2, Wr2, g2, be2,
           Wl3, bl3, Wr3, g3, be3, W1, b1, W2, b2, edge_index, batch):
    n, d = x.shape
    h = W_emb.shape[1]
    c = W2.shape[1]
    src = edge_index[0]
    dst = edge_index[1]

    hh = _embed(x, W_emb, b_emb)

    cntp = _sc_count(dst, n, h)
    aggp = _sc_aggregate(hh, src, dst)
    t, mom = _sage_dense(aggp, cntp, hh, Wl1, bl1, Wr1)
    m1 = _bn_relu(t, mom, g1, be1)

    aggp2 = _sc_aggregate(m1, src, dst)
    t, mom = _sage_dense(aggp2, cntp, m1, Wl2, bl2, Wr2)
    m2 = _bn_relu(t, mom, g2, be2)

    aggp3 = _sc_aggregate(m2, src, dst)
    t, mom = _sage_dense(aggp3, cntp, m2, Wl3, bl3, Wr3)
    m3 = _bn_relu(t, mom, g3, be3)

    num_groups = 512
    nb = n // _BR
    batch3 = batch.reshape(nb, 1, _BR)
    w2p = jnp.zeros((h, h), jnp.float32).at[:, :c].set(W2)
    b2p = jnp.zeros((1, h), jnp.float32).at[0, :c].set(b2)
    out_pad = _pool_head(m3, batch3, W1, b1, w2p, b2p, num_groups)
    out = out_pad[:, :c]
    return (out, m1, m2, m3)


# speedup vs baseline: 10.5302x; 1.1124x over previous
import functools

import jax
import jax.numpy as jnp
from jax import lax
from jax.experimental import pallas as pl
from jax.experimental.pallas import tpu as pltpu
from jax.experimental.pallas import tpu_sc as plsc

_NC = 2
_NS = 16
_CH = 80
_SB = 25


def _pad_rows(n):
    q = _NS * _CH
    return ((n + q - 1) // q) * q


def _sc_aggregate(h, src, dst):
    n, d = h.shape
    e = src.shape[0]
    nw = _NC * _NS
    epw = e // nw
    nch = epw // _CH
    np_ = _pad_rows(n)
    rpw = np_ // _NS
    lanes = d // 16

    nsb = nch // _SB

    scratch = [
        pltpu.VMEM((_SB * _CH,), jnp.int32),
        pltpu.VMEM((_SB, _CH), jnp.int32),
        pltpu.VMEM((_CH, d), jnp.float32),
        pltpu.VMEM((_CH, d), jnp.float32),
        pltpu.VMEM((_CH, d), jnp.float32),
        pltpu.VMEM_SHARED((np_, d), jnp.float32),
        pltpu.SemaphoreType.DMA,
        pltpu.SemaphoreType.DMA,
        pltpu.SemaphoreType.DMA,
        pltpu.SemaphoreType.DMA,
        pltpu.SemaphoreType.DMA,
        pltpu.SemaphoreType.DMA,
    ]

    def body(h_hbm, src_hbm, dst_hbm, agg_out, src_v, dst_v, buf0, buf1,
             buf2, agg_sh, sg0, sg1, sg2, ss0, ss1, ss2):
        c = lax.axis_index("c")
        s = lax.axis_index("s")
        wid = c * _NS + s
        bufs = (buf0, buf1, buf2)
        sgs = (sg0, sg1, sg2)
        sss = (ss0, ss1, ss2)

        def fill_z(i, carry):
            buf0[i // lanes, pl.ds((i % lanes) * 16, 16)] = jnp.zeros(
                (16,), jnp.float32)
            return carry
        lax.fori_loop(0, _CH * lanes, fill_z, 0)

        r0 = s * rpw
        for j in range(rpw // _CH):
            pltpu.sync_copy(buf0, agg_sh.at[pl.ds(r0 + j * _CH, _CH), :])
        plsc.subcore_barrier()

        def gather(j, p):
            return pltpu.make_async_copy(
                h_hbm.at[src_v.at[pl.ds(j * _CH, _CH)]], bufs[p], sgs[p])

        def scatter(j, p):
            return pltpu.async_copy(bufs[p], agg_sh.at[dst_v.at[j]],
                                    sss[p], add=True)

        def super_step(k, carry):
            pltpu.sync_copy(src_hbm.at[wid, k], src_v)
            pltpu.sync_copy(dst_hbm.at[wid, k], dst_v)
            gather(0, 0).start()
            gather(1, 1).start()
            pend = {}
            for j in range(_SB):
                p = j % 3
                if j + 2 < _SB:
                    pm = (j + 2) % 3
                    if pm in pend:
                        pend.pop(pm).wait()
                    gather(j + 2, pm).start()
                gather(j, p).wait()
                pend[p] = scatter(j, p)
            for p in list(pend):
                pend.pop(p).wait()
            return carry
        lax.fori_loop(0, nsb, super_step, 0)

        plsc.subcore_barrier()
        pltpu.sync_copy(agg_sh.at[pl.ds(r0, rpw), :],
                        agg_out.at[c, pl.ds(r0, rpw), :])

    mesh = plsc.VectorSubcoreMesh(core_axis_name="c", subcore_axis_name="s")
    fn = pl.kernel(body,
                   out_type=jax.ShapeDtypeStruct((_NC, np_, d), jnp.float32),
                   mesh=mesh, scratch_types=scratch)
    nw_ = _NC * _NS
    srcr = src.reshape(nw_, nsb, _SB * _CH)
    dstr = dst.reshape(nw_, nsb, _SB, _CH)
    return fn(h, srcr, dstr)


def _sc_count(dst, n, d):
    e = dst.shape[0]
    nw = _NC * _NS
    epw = e // nw
    nch = epw // _CH
    np_ = _pad_rows(n)
    rpw = np_ // _NS
    lanes = d // 16

    nsb = nch // _SB

    scratch = [
        pltpu.VMEM((_SB, _CH), jnp.int32),
        pltpu.VMEM((_CH, d), jnp.float32),
        pltpu.VMEM_SHARED((np_, d), jnp.float32),
        pltpu.SemaphoreType.DMA,
    ]

    def body(dst_hbm, cnt_out, dst_v, ones_v, cnt_sh, sem):
        c = lax.axis_index("c")
        s = lax.axis_index("s")
        wid = c * _NS + s

        def fill(val):
            def f(i, carry):
                ones_v[i // lanes, pl.ds((i % lanes) * 16, 16)] = jnp.full(
                    (16,), val, jnp.float32)
                return carry
            lax.fori_loop(0, _CH * lanes, f, 0)

        fill(0.0)
        r0 = s * rpw
        for j in range(rpw // _CH):
            pltpu.sync_copy(ones_v, cnt_sh.at[pl.ds(r0 + j * _CH, _CH), :])
        fill(1.0)
        plsc.subcore_barrier()

        def super_step(k, carry):
            pltpu.sync_copy(dst_hbm.at[wid, k], dst_v)
            descs = [pltpu.async_copy(ones_v, cnt_sh.at[dst_v.at[j]], sem,
                                      add=True)
                     for j in range(_SB)]
            for dsc in descs:
                dsc.wait()
            return carry
        lax.fori_loop(0, nsb, super_step, 0)

        plsc.subcore_barrier()
        pltpu.sync_copy(cnt_sh.at[pl.ds(r0, rpw), :],
                        cnt_out.at[c, pl.ds(r0, rpw), :])

    mesh = plsc.VectorSubcoreMesh(core_axis_name="c", subcore_axis_name="s")
    fn = pl.kernel(body,
                   out_type=jax.ShapeDtypeStruct((_NC, np_, d), jnp.float32),
                   mesh=mesh, scratch_types=scratch)
    return fn(dst.reshape(_NC * _NS, nsb, _SB, _CH))


_BR = 1000


def _embed(x, w, b):
    n, d = x.shape
    h = w.shape[1]

    def body(x_ref, w_ref, b_ref, o_ref):
        o_ref[...] = jnp.dot(x_ref[...], w_ref[...],
                             preferred_element_type=jnp.float32) + b_ref[...]

    return pl.pallas_call(
        body,
        grid=(n // _BR,),
        in_specs=[
            pl.BlockSpec((_BR, d), lambda i: (i, 0)),
            pl.BlockSpec((d, h), lambda i: (0, 0)),
            pl.BlockSpec((1, h), lambda i: (0, 0)),
        ],
        out_specs=pl.BlockSpec((_BR, h), lambda i: (i, 0)),
        out_shape=jax.ShapeDtypeStruct((n, h), jnp.float32),
    )(x, w, b.reshape(1, h))


def _sage_dense(aggp, cntp, h_in, wl, bl, wr):
    n, h = h_in.shape
    nb = n // _BR

    def body(a_ref, c_ref, h_ref, wl_ref, bl_ref, wr_ref, t_ref, m_ref):
        i = pl.program_id(0)
        agg = a_ref[0] + a_ref[1]
        cnt = c_ref[0] + c_ref[1]
        mean = agg / jnp.maximum(cnt, 1.0)
        t = (jnp.dot(mean, wl_ref[...], preferred_element_type=jnp.float32)
             + bl_ref[...]
             + jnp.dot(h_ref[...], wr_ref[...],
                       preferred_element_type=jnp.float32))
        t_ref[...] = t
        mom = jnp.concatenate(
            [jnp.sum(t, axis=0)[None], jnp.sum(t * t, axis=0)[None],
             jnp.zeros((6, h), jnp.float32)], axis=0)

        @pl.when(i == 0)
        def _():
            m_ref[...] = mom

        @pl.when(i > 0)
        def _():
            m_ref[...] += mom

    return pl.pallas_call(
        body,
        grid=(nb,),
        in_specs=[
            pl.BlockSpec((_NC, _BR, h), lambda i: (0, i, 0)),
            pl.BlockSpec((_NC, _BR, h), lambda i: (0, i, 0)),
            pl.BlockSpec((_BR, h), lambda i: (i, 0)),
            pl.BlockSpec((h, h), lambda i: (0, 0)),
            pl.BlockSpec((1, h), lambda i: (0, 0)),
            pl.BlockSpec((h, h), lambda i: (0, 0)),
        ],
        out_specs=[
            pl.BlockSpec((_BR, h), lambda i: (i, 0)),
            pl.BlockSpec((8, h), lambda i: (0, 0)),
        ],
        out_shape=[
            jax.ShapeDtypeStruct((n, h), jnp.float32),
            jax.ShapeDtypeStruct((8, h), jnp.float32),
        ],
    )(aggp, cntp, h_in, wl, bl.reshape(1, h), wr)


def _bn_relu(t, mom, g, be):
    n, h = t.shape
    inv_n = 1.0 / n

    def body(t_ref, m_ref, g_ref, be_ref, o_ref):
        mu = m_ref[0:1, :] * inv_n
        var = m_ref[1:2, :] * inv_n - mu * mu
        scale = g_ref[...] * lax.rsqrt(var + 1e-5)
        o_ref[...] = jnp.maximum((t_ref[...] - mu) * scale + be_ref[...], 0.0)

    return pl.pallas_call(
        body,
        grid=(n // _BR,),
        in_specs=[
            pl.BlockSpec((_BR, h), lambda i: (i, 0)),
            pl.BlockSpec((8, h), lambda i: (0, 0)),
            pl.BlockSpec((1, h), lambda i: (0, 0)),
            pl.BlockSpec((1, h), lambda i: (0, 0)),
        ],
        out_specs=pl.BlockSpec((_BR, h), lambda i: (i, 0)),
        out_shape=jax.ShapeDtypeStruct((n, h), jnp.float32),
    )(t, mom, g.reshape(1, h), be.reshape(1, h))


def _pool_head(hm, batch3, w1, b1, w2p, b2p, num_groups):
    n, h = hm.shape
    nb = n // _BR
    g = num_groups

    def body(h_ref, b_ref, w1_ref, b1_ref, w2_ref, b2_ref, o_ref,
             acc, cacc):
        i = pl.program_id(0)
        bvec = b_ref[0, 0, :]
        seg = lax.broadcasted_iota(jnp.int32, (g, _BR), 0)
        p = (seg == bvec[None, :]).astype(jnp.float32)
        pooled = jnp.dot(p, h_ref[...], preferred_element_type=jnp.float32)
        csum = jnp.broadcast_to(jnp.sum(p, axis=1)[None, :], (8, g))

        @pl.when(i == 0)
        def _():
            acc[...] = pooled
            cacc[...] = csum

        @pl.when(i > 0)
        def _():
            acc[...] += pooled
            cacc[...] += csum

        @pl.when(i == nb - 1)
        def _():
            inv = 1.0 / jnp.maximum(cacc[0:1, :], 1.0)
            pm = acc[...] * inv.reshape(g, 1)
            z = jnp.maximum(
                jnp.dot(pm, w1_ref[...], preferred_element_type=jnp.float32)
                + b1_ref[...], 0.0)
            o_ref[...] = jnp.dot(
                z, w2_ref[...], preferred_element_type=jnp.float32) + b2_ref[...]

    return pl.pallas_call(
        body,
        grid=(nb,),
        in_specs=[
            pl.BlockSpec((_BR, h), lambda i: (i, 0)),
            pl.BlockSpec((1, 1, _BR), lambda i: (i, 0, 0)),
            pl.BlockSpec((h, h), lambda i: (0, 0)),
            pl.BlockSpec((1, h), lambda i: (0, 0)),
            pl.BlockSpec((h, h), lambda i: (0, 0)),
            pl.BlockSpec((1, h), lambda i: (0, 0)),
        ],
        out_specs=pl.BlockSpec((g, h), lambda i: (0, 0)),
        out_shape=jax.ShapeDtypeStruct((g, h), jnp.float32),
        scratch_shapes=[
            pltpu.VMEM((g, h), jnp.float32),
            pltpu.VMEM((8, g), jnp.float32),
        ],
    )(hm, batch3, w1, b1.reshape(1, h), w2p, b2p)


def kernel(x, W_emb, b_emb, Wl1, bl1, Wr1, g1, be1, Wl2, bl2, Wr2, g2, be2,
           Wl3, bl3, Wr3, g3, be3, W1, b1, W2, b2, edge_index, batch):
    n, d = x.shape
    h = W_emb.shape[1]
    c = W2.shape[1]
    src = edge_index[0]
    dst = edge_index[1]

    hh = _embed(x, W_emb, b_emb)

    cntp = _sc_count(dst, n, h)
    aggp = _sc_aggregate(hh, src, dst)
    t, mom = _sage_dense(aggp, cntp, hh, Wl1, bl1, Wr1)
    m1 = _bn_relu(t, mom, g1, be1)

    aggp2 = _sc_aggregate(m1, src, dst)
    t, mom = _sage_dense(aggp2, cntp, m1, Wl2, bl2, Wr2)
    m2 = _bn_relu(t, mom, g2, be2)

    aggp3 = _sc_aggregate(m2, src, dst)
    t, mom = _sage_dense(aggp3, cntp, m2, Wl3, bl3, Wr3)
    m3 = _bn_relu(t, mom, g3, be3)

    num_groups = 512
    nb = n // _BR
    batch3 = batch.reshape(nb, 1, _BR)
    w2p = jnp.zeros((h, h), jnp.float32).at[:, :c].set(W2)
    b2p = jnp.zeros((1, h), jnp.float32).at[0, :c].set(b2)
    out_pad = _pool_head(m3, batch3, W1, b1, w2p, b2p, num_groups)
    out = out_pad[:, :c]
    return (out, m1, m2, m3)
